# baseline (speedup 1.0000x reference)
import functools

import jax
import jax.numpy as jnp
from jax import lax
from jax.experimental import pallas as pl
from jax.experimental.pallas import tpu as pltpu
from jax.experimental.pallas import tpu_sc as plsc

_B = 4096
_S = 200
_D = 64
_DW = _D // 2
_NW = 32
_TOTAL = _B * _S

_GRP = 128
_NGRP = 8
_CHUNK = _GRP * _NGRP
_ROWS_PER_W = _TOTAL // _NW
_NCHUNK = _ROWS_PER_W // _CHUNK


def _gather_kernel(idx_hbm, tok_hbm, out_hbm, idx_v, rows_v, sem):
    wid = lax.axis_index("s") * 2 + lax.axis_index("c")

    def chunk_body(c, carry):
        base = pl.multiple_of(wid * _ROWS_PER_W + c * _CHUNK, _CHUNK)

        irow = pl.multiple_of(base // _GRP, _NGRP)
        pltpu.sync_copy(idx_hbm.at[pl.ds(irow, _NGRP)], idx_v)

        copies = []
        for j in range(_NGRP):
            copies.append(
                pltpu.async_copy(
                    tok_hbm.at[idx_v.at[j]],
                    rows_v.at[pl.ds(j * _GRP, _GRP)],
                    sem,
                )
            )
        for cp in copies:
            cp.wait()

        pltpu.sync_copy(rows_v, out_hbm.at[pl.ds(base, _CHUNK)])
        return carry

    lax.fori_loop(0, _NCHUNK, chunk_body, 0)


def _add_kernel(g_ref, pos_ref, o_ref):
    g = g_ref[...].reshape(_B, _DW)
    t = jnp.swapaxes(g, 0, 1)
    f = pltpu.bitcast(t, jnp.bfloat16)
    s = pl.program_id(0)
    p8 = jnp.swapaxes(pos_ref[...], 0, 1)
    lane = jax.lax.broadcasted_iota(jnp.int32, (_D, 8), 1)
    p = jnp.sum(
        jnp.where(lane == lax.rem(s, 8), p8, jnp.bfloat16(0)),
        axis=1,
        keepdims=True,
    )
    o_ref[...] = (f + p).reshape(1, _D, _B)


@jax.jit
def kernel(x, token_table, pos_table):
    tok_i32 = lax.bitcast_convert_type(
        token_table.reshape(token_table.shape[0], _DW, 2), jnp.int32
    )
    idx = x.T.reshape(_TOTAL // _GRP, _GRP).astype(jnp.int32)
    pos_b = pos_table[:_S]

    mesh = plsc.VectorSubcoreMesh(core_axis_name="c", subcore_axis_name="s")
    g = pl.kernel(
        _gather_kernel,
        mesh=mesh,
        compiler_params=pltpu.CompilerParams(use_tc_tiling_on_sc=False),
        out_type=jax.ShapeDtypeStruct((_TOTAL, _DW), jnp.int32),
        scratch_types=[
            pltpu.VMEM((_NGRP, _GRP), jnp.int32),
            pltpu.VMEM((_CHUNK, _DW), jnp.int32),
            pltpu.SemaphoreType.DMA,
        ],
    )(idx, tok_i32)

    g3 = g.reshape(_S, _B, _DW)
    out = pl.pallas_call(
        _add_kernel,
        grid=(_S,),
        in_specs=[
            pl.BlockSpec((1, _B, _DW), lambda s: (s, 0, 0)),
            pl.BlockSpec((8, _D), lambda s: (s // 8, 0)),
        ],
        out_specs=pl.BlockSpec((1, _D, _B), lambda s: (s, 0, 0)),
        out_shape=jax.ShapeDtypeStruct((_S, _D, _B), jnp.bfloat16),
    )(g3, pos_b)

    return out.transpose(2, 0, 1)
